# baseline (speedup 1.0000x reference)
import jax
import jax.numpy as jnp
from jax import lax
from jax.experimental import pallas as pl
from jax.experimental.pallas import tpu as pltpu
from jax.experimental.pallas import tpu_sc as plsc

N_ROWS = 800000
N_AE = 32
N_DE = 16
N_SEG_ATOM = 50000

B = 1280
NB = N_ROWS // B
W = 128
TR = 50432
TE = 128
FS = 40
FD = 56
KC = 8
RC = TR // KC

NC = 2
NS = 16
NW = NC * NS
GC = 128
CPW = 196
NP = NW * CPW * GC

_C00 = (((0,), (0,)), ((), ()))
_C11 = (((1,), (1,)), ((), ()))
_F32 = jnp.float32


def _k1a_body(lo_ref, hi_ref, x_ref, dist_ref, aidx_ref, eidx_ref,
              aacc_ref, eacc_ref, c_ref, dd_ref):
    i = pl.program_id(0)

    @pl.when(i == 0)
    def _():
        aacc_ref[...] = jnp.zeros((TR, FD), _F32)
        eacc_ref[...] = jnp.zeros((TE, FD), _F32)
        c_ref[...] = jnp.zeros((TR, TE), _F32)
        dd_ref[...] = jnp.zeros((24, 16), _F32)

    aidx_row = aidx_ref[0]
    eidx_row = eidx_ref[0]

    dist = dist_ref[...]
    ones8 = jnp.ones((B, 8), _F32)
    x56a = jnp.concatenate([x_ref[:, 0:N_AE], ones8, dist], axis=1)
    x56e = jnp.concatenate([ones8, x_ref[:, N_AE:2 * N_AE], dist], axis=1)

    sub_e = lax.broadcasted_iota(jnp.int32, (TE, B), 0)
    ohe = (sub_e == eidx_row).astype(_F32)
    eacc_ref[...] += jnp.dot(ohe, x56e, preferred_element_type=_F32)

    dext = jnp.concatenate([dist, ones8], axis=1)
    dd_ref[...] += lax.dot_general(dext, dist, _C00,
                                   preferred_element_type=_F32)

    lo = lo_ref[i]
    hi = hi_ref[i]
    base = (lo // 8) * 8
    nwin = (hi - base) // W + 1
    sub_a = lax.broadcasted_iota(jnp.int32, (W, B), 0)
    rel0 = aidx_row - base

    def wloop(k, _):
        oh = (sub_a == (rel0 - k * W)).astype(_F32)
        aacc_ref[pl.ds(base + k * W, W), :] += jnp.dot(
            oh, x56a, preferred_element_type=_F32)
        c_ref[pl.ds(base + k * W, W), :] += lax.dot_general(
            oh, ohe, _C11, preferred_element_type=_F32)
        return 0

    lax.fori_loop(0, nwin, wloop, 0)


def _pooled(tab, cnt_col, val_cols):
    n = jnp.maximum(tab[:, cnt_col:cnt_col + 1], 1.0)
    p = jnp.maximum(tab[:, val_cols:val_cols + N_AE], 0.0) / n
    return n, p


def _k1b_body(aacc_ref, c_ref, eacc_ref, dd_ref, w1_ref, gamma_ref, beta_ref,
              sl_ref, pa_ref, acc_ref):
    i = pl.program_id(0)

    @pl.when(i == 0)
    def _():
        acc_ref[...] = jnp.zeros((88, 32), _F32)

    tab = aacc_ref[...]
    na, pa = _pooled(tab, N_AE, 0)
    pa_ref[...] = pa
    paw = pa * na
    onesc = jnp.ones((RC, 1), _F32)

    eacc = eacc_ref[...]
    ne, pe = _pooled(eacc, 0, 8)

    cpe = jnp.dot(c_ref[...], pe, preferred_element_type=_F32)

    acc_ref[0:32, :] += lax.dot_general(pa, paw, _C00,
                                        preferred_element_type=_F32)
    acc_ref[32:64, :] += lax.dot_general(pa, cpe, _C00,
                                         preferred_element_type=_F32)
    acc_ref[64:80, :] += lax.dot_general(tab[:, FS:FD], pa, _C00,
                                         preferred_element_type=_F32)
    acc_ref[80:81, :] += lax.dot_general(onesc, paw, _C00,
                                         preferred_element_type=_F32)

    @pl.when(i == KC - 1)
    def _():
        qa = acc_ref[0:32, :]
        gae = acc_ref[32:64, :]
        gda = acc_ref[64:80, :]
        sa = acc_ref[80:81, :]

        pew = pe * ne
        onese = jnp.ones((TE, 1), _F32)
        qe = lax.dot_general(pe, pew, _C00, preferred_element_type=_F32)
        gde = lax.dot_general(eacc[:, FS:FD], pe, _C00,
                              preferred_element_type=_F32)
        se = lax.dot_general(onese, pew, _C00, preferred_element_type=_F32)

        gdd = dd_ref[0:16, :]
        sd = dd_ref[16:17, :]

        w1d = w1_ref[0:N_DE, :]
        w1a = w1_ref[N_DE:N_DE + N_AE, :]
        w1e = w1_ref[N_DE + N_AE:, :]

        mu = (jnp.dot(sd, w1d, preferred_element_type=_F32)
              + jnp.dot(sa, w1a, preferred_element_type=_F32)
              + jnp.dot(se, w1e, preferred_element_type=_F32)) * (1.0 / N_ROWS)

        def dsum(a, m, b):
            return jnp.sum(a * jnp.dot(m, b, preferred_element_type=_F32),
                           axis=0, keepdims=True)

        hh = (dsum(w1d, gdd, w1d)
              + 2.0 * dsum(w1d, gda, w1a)
              + 2.0 * dsum(w1d, gde, w1e)
              + dsum(w1a, qa, w1a)
              + 2.0 * dsum(w1a, gae, w1e)
              + dsum(w1e, qe, w1e))

        var = hh * (1.0 / N_ROWS) - mu * mu
        scale = gamma_ref[...] * lax.rsqrt(var + 1e-5)
        shift = beta_ref[...] - mu * scale
        sl_ref[...] = jnp.concatenate(
            [scale, shift, jnp.zeros((6, 64), _F32)], axis=0)


def _ksc_body(pa_hbm, idx_hbm, out_hbm, idx_v, rows_v, sem):
    wid = lax.axis_index("s") * NC + lax.axis_index("c")
    base0 = wid * (CPW * GC)

    def chunk(c, carry):
        base = pl.multiple_of(base0 + c * GC, GC)
        pltpu.sync_copy(idx_hbm.at[pl.ds(base, GC)], idx_v)
        pltpu.async_copy(pa_hbm.at[idx_v], rows_v, sem).wait()
        pltpu.sync_copy(rows_v, out_hbm.at[pl.ds(base, GC)])
        return carry

    lax.fori_loop(0, CPW, chunk, 0)


def _k2_body(eacc_ref, dist_ref, pa_ref, eidx_ref, w1_ref, x_ref, sl_ref,
             out_ref):
    eidx_row = eidx_ref[0]

    ecnt = jnp.maximum(eacc_ref[:, 0:1], 1.0)
    pe_tab = jnp.maximum(eacc_ref[:, 8:FS], 0.0) / ecnt
    sub_e = lax.broadcasted_iota(jnp.int32, (TE, B), 0)
    ohe = (sub_e == eidx_row).astype(_F32)
    pe_t = lax.dot_general(pe_tab, ohe, _C00, preferred_element_type=_F32)
    pe = pe_t.T

    hb = (jnp.dot(dist_ref[...], w1_ref[0:N_DE, :], preferred_element_type=_F32)
          + jnp.dot(pa_ref[...], w1_ref[N_DE:N_DE + N_AE, :],
                    preferred_element_type=_F32)
          + jnp.dot(pe, w1_ref[N_DE + N_AE:, :], preferred_element_type=_F32))

    scale = sl_ref[0:1, :]
    shift = sl_ref[1:2, :]
    out_ref[...] = jnp.maximum(hb * scale + shift + x_ref[...], 0.0)


@jax.jit
def kernel(x, dist_feat, atom_idx, ele_idx, W1, b1, gamma, beta):
    del b1
    aidx = atom_idx.astype(jnp.int32)
    eidx = ele_idx.astype(jnp.int32)
    lo = aidx[::B]
    hi = aidx[B - 1::B]
    aidx3 = aidx.reshape(NB, 1, B)
    eidx3 = eidx.reshape(NB, 1, B)
    aidx_p = jnp.pad(aidx, (0, NP - N_ROWS))

    grid_a = pltpu.PrefetchScalarGridSpec(
        num_scalar_prefetch=2,
        grid=(NB,),
        in_specs=[
            pl.BlockSpec((B, 64), lambda i, lo, hi: (i, 0)),
            pl.BlockSpec((B, N_DE), lambda i, lo, hi: (i, 0)),
            pl.BlockSpec((1, 1, B), lambda i, lo, hi: (i, 0, 0)),
            pl.BlockSpec((1, 1, B), lambda i, lo, hi: (i, 0, 0)),
        ],
        out_specs=[
            pl.BlockSpec((TR, FD), lambda i, lo, hi: (0, 0)),
            pl.BlockSpec((TE, FD), lambda i, lo, hi: (0, 0)),
            pl.BlockSpec((TR, TE), lambda i, lo, hi: (0, 0)),
            pl.BlockSpec((24, 16), lambda i, lo, hi: (0, 0)),
        ],
    )
    aacc, eacc, ctab, dd = pl.pallas_call(
        _k1a_body,
        grid_spec=grid_a,
        out_shape=[
            jax.ShapeDtypeStruct((TR, FD), _F32),
            jax.ShapeDtypeStruct((TE, FD), _F32),
            jax.ShapeDtypeStruct((TR, TE), _F32),
            jax.ShapeDtypeStruct((24, 16), _F32),
        ],
        compiler_params=pltpu.CompilerParams(
            dimension_semantics=("arbitrary",),
        ),
    )(lo, hi, x, dist_feat, aidx3, eidx3)

    sl, pa_tab = pl.pallas_call(
        _k1b_body,
        grid=(KC,),
        in_specs=[
            pl.BlockSpec((RC, FD), lambda i: (i, 0)),
            pl.BlockSpec((RC, TE), lambda i: (i, 0)),
            pl.BlockSpec((TE, FD), lambda i: (0, 0)),
            pl.BlockSpec((24, 16), lambda i: (0, 0)),
            pl.BlockSpec((80, 64), lambda i: (0, 0)),
            pl.BlockSpec((1, 64), lambda i: (0, 0)),
            pl.BlockSpec((1, 64), lambda i: (0, 0)),
        ],
        out_specs=[
            pl.BlockSpec((8, 64), lambda i: (0, 0)),
            pl.BlockSpec((RC, N_AE), lambda i: (i, 0)),
        ],
        out_shape=[
            jax.ShapeDtypeStruct((8, 64), _F32),
            jax.ShapeDtypeStruct((TR, N_AE), _F32),
        ],
        scratch_shapes=[pltpu.VMEM((88, 32), _F32)],
        compiler_params=pltpu.CompilerParams(
            dimension_semantics=("arbitrary",),
        ),
    )(aacc, ctab, eacc, dd, W1, gamma.reshape(1, 64), beta.reshape(1, 64))

    sc_gather = pl.kernel(
        _ksc_body,
        out_type=jax.ShapeDtypeStruct((NP, N_AE), _F32),
        mesh=plsc.VectorSubcoreMesh(core_axis_name="c", subcore_axis_name="s",
                                    num_cores=NC, num_subcores=NS),
        scratch_types=[
            pltpu.VMEM((GC,), jnp.int32),
            pltpu.VMEM((GC, N_AE), _F32),
            pltpu.SemaphoreType.DMA,
        ],
        compiler_params=pltpu.CompilerParams(use_tc_tiling_on_sc=False),
    )
    parows = sc_gather(pa_tab, aidx_p)

    (out,) = pl.pallas_call(
        _k2_body,
        grid=(NB,),
        in_specs=[
            pl.BlockSpec((TE, FD), lambda i: (0, 0)),
            pl.BlockSpec((B, N_DE), lambda i: (i, 0)),
            pl.BlockSpec((B, N_AE), lambda i: (i, 0)),
            pl.BlockSpec((1, 1, B), lambda i: (i, 0, 0)),
            pl.BlockSpec((80, 64), lambda i: (0, 0)),
            pl.BlockSpec((B, 64), lambda i: (i, 0)),
            pl.BlockSpec((8, 64), lambda i: (0, 0)),
        ],
        out_specs=[
            pl.BlockSpec((B, 64), lambda i: (i, 0)),
        ],
        out_shape=[
            jax.ShapeDtypeStruct((N_ROWS, 64), jnp.float32),
        ],
        compiler_params=pltpu.CompilerParams(
            dimension_semantics=("arbitrary",),
        ),
    )(eacc, dist_feat, parows, eidx3, W1, x, sl)
    return out


# speedup vs baseline: 1.0777x; 1.0777x over previous
import jax
import jax.numpy as jnp
from jax import lax
from jax.experimental import pallas as pl
from jax.experimental.pallas import tpu as pltpu
from jax.experimental.pallas import tpu_sc as plsc

N_ROWS = 800000
N_AE = 32
N_DE = 16
N_SEG_ATOM = 50000

B = 1280
NB = N_ROWS // B
W = 128
TR = 50432
TE = 128
FS = 40
FD = 56
KC = 8
RC = TR // KC

NC = 2
NS = 16
NW = NC * NS
GC = 128
CPW = 196
NP = NW * CPW * GC

_C00 = (((0,), (0,)), ((), ()))
_C11 = (((1,), (1,)), ((), ()))
_F32 = jnp.float32


def _k1a_body(lo_ref, hi_ref, x_ref, dist_ref, aidx_ref, eidx_ref,
              aacc_ref, eacc_ref, c_ref, dd_ref):
    i = pl.program_id(0)

    @pl.when(i == 0)
    def _():
        aacc_ref[...] = jnp.zeros((TR, FD), _F32)
        eacc_ref[...] = jnp.zeros((TE, FD), _F32)
        c_ref[...] = jnp.zeros((TR, TE), _F32)
        dd_ref[...] = jnp.zeros((24, 16), _F32)

    aidx_row = aidx_ref[0]
    eidx_row = eidx_ref[0]

    dist = dist_ref[...]
    ones8 = jnp.ones((B, 8), _F32)
    x56a = jnp.concatenate([x_ref[:, 0:N_AE], ones8, dist], axis=1)
    x56e = jnp.concatenate([ones8, x_ref[:, N_AE:2 * N_AE], dist], axis=1)

    sub_e = lax.broadcasted_iota(jnp.int32, (TE, B), 0)
    ohe = (sub_e == eidx_row).astype(_F32)
    eacc_ref[...] += jnp.dot(ohe, x56e, preferred_element_type=_F32)

    dext = jnp.concatenate([dist, ones8], axis=1)
    dd_ref[...] += lax.dot_general(dext, dist, _C00,
                                   preferred_element_type=_F32)

    lo = lo_ref[i]
    hi = hi_ref[i]
    base = (lo // 8) * 8
    nwin = (hi - base) // W + 1
    sub_a = lax.broadcasted_iota(jnp.int32, (W, B), 0)
    rel0 = aidx_row - base

    def wloop(k, _):
        oh = (sub_a == (rel0 - k * W)).astype(_F32)
        aacc_ref[pl.ds(base + k * W, W), :] += jnp.dot(
            oh, x56a, preferred_element_type=_F32)
        c_ref[pl.ds(base + k * W, W), :] += lax.dot_general(
            oh, ohe, _C11, preferred_element_type=_F32)
        return 0

    lax.fori_loop(0, nwin, wloop, 0)


def _pooled(tab, cnt_col, val_cols):
    n = jnp.maximum(tab[:, cnt_col:cnt_col + 1], 1.0)
    p = jnp.maximum(tab[:, val_cols:val_cols + N_AE], 0.0) / n
    return n, p


def _k1b_body(aacc_ref, c_ref, eacc_ref, dd_ref, w1_ref, gamma_ref, beta_ref,
              sl_ref, pa_ref, acc_ref):
    i = pl.program_id(0)

    @pl.when(i == 0)
    def _():
        acc_ref[...] = jnp.zeros((88, 32), _F32)

    tab = aacc_ref[...]
    na, pa = _pooled(tab, N_AE, 0)
    pa_ref[...] = pa
    paw = pa * na
    onesc = jnp.ones((RC, 1), _F32)

    eacc = eacc_ref[...]
    ne, pe = _pooled(eacc, 0, 8)

    cpe = jnp.dot(c_ref[...], pe, preferred_element_type=_F32)

    acc_ref[0:32, :] += lax.dot_general(pa, paw, _C00,
                                        preferred_element_type=_F32)
    acc_ref[32:64, :] += lax.dot_general(pa, cpe, _C00,
                                         preferred_element_type=_F32)
    acc_ref[64:80, :] += lax.dot_general(tab[:, FS:FD], pa, _C00,
                                         preferred_element_type=_F32)
    acc_ref[80:81, :] += lax.dot_general(onesc, paw, _C00,
                                         preferred_element_type=_F32)

    @pl.when(i == KC - 1)
    def _():
        qa = acc_ref[0:32, :]
        gae = acc_ref[32:64, :]
        gda = acc_ref[64:80, :]
        sa = acc_ref[80:81, :]

        pew = pe * ne
        onese = jnp.ones((TE, 1), _F32)
        qe = lax.dot_general(pe, pew, _C00, preferred_element_type=_F32)
        gde = lax.dot_general(eacc[:, FS:FD], pe, _C00,
                              preferred_element_type=_F32)
        se = lax.dot_general(onese, pew, _C00, preferred_element_type=_F32)

        gdd = dd_ref[0:16, :]
        sd = dd_ref[16:17, :]

        w1d = w1_ref[0:N_DE, :]
        w1a = w1_ref[N_DE:N_DE + N_AE, :]
        w1e = w1_ref[N_DE + N_AE:, :]

        mu = (jnp.dot(sd, w1d, preferred_element_type=_F32)
              + jnp.dot(sa, w1a, preferred_element_type=_F32)
              + jnp.dot(se, w1e, preferred_element_type=_F32)) * (1.0 / N_ROWS)

        def dsum(a, m, b):
            return jnp.sum(a * jnp.dot(m, b, preferred_element_type=_F32),
                           axis=0, keepdims=True)

        hh = (dsum(w1d, gdd, w1d)
              + 2.0 * dsum(w1d, gda, w1a)
              + 2.0 * dsum(w1d, gde, w1e)
              + dsum(w1a, qa, w1a)
              + 2.0 * dsum(w1a, gae, w1e)
              + dsum(w1e, qe, w1e))

        var = hh * (1.0 / N_ROWS) - mu * mu
        scale = gamma_ref[...] * lax.rsqrt(var + 1e-5)
        shift = beta_ref[...] - mu * scale
        sl_ref[...] = jnp.concatenate(
            [scale, shift, jnp.zeros((6, 64), _F32)], axis=0)


def _ksc_body(pa_hbm, idx_hbm, out_hbm, idx_v, rows_v, sem0, sem1):
    wid = lax.axis_index("s") * NC + lax.axis_index("c")
    base0 = pl.multiple_of(wid * (CPW * GC), GC)
    pltpu.sync_copy(idx_hbm.at[pl.ds(base0, CPW * GC)], idx_v)

    sems = (sem0, sem1)

    def fire(c, b):
        pltpu.async_copy(
            pa_hbm.at[idx_v.at[pl.ds(pl.multiple_of(c * GC, GC), GC)]],
            rows_v.at[b], sems[b])

    fire(0, 0)
    fire(1, 1)

    def step(g, carry):
        for b in range(2):
            c = g * 2 + b
            pltpu.make_async_copy(pa_hbm.at[idx_v.at[pl.ds(0, GC)]],
                                  rows_v.at[b], sems[b]).wait()
            pltpu.sync_copy(rows_v.at[b],
                            out_hbm.at[pl.ds(base0 + c * GC, GC)])

            @pl.when(c + 2 < CPW)
            def _():
                fire(c + 2, b)
        return carry

    lax.fori_loop(0, CPW // 2, step, 0)


def _k2_body(eacc_ref, dist_ref, pa_ref, eidx_ref, w1_ref, x_ref, sl_ref,
             out_ref):
    eidx_row = eidx_ref[0]

    ecnt = jnp.maximum(eacc_ref[:, 0:1], 1.0)
    pe_tab = jnp.maximum(eacc_ref[:, 8:FS], 0.0) / ecnt
    sub_e = lax.broadcasted_iota(jnp.int32, (TE, B), 0)
    ohe = (sub_e == eidx_row).astype(_F32)
    pe_t = lax.dot_general(pe_tab, ohe, _C00, preferred_element_type=_F32)
    pe = pe_t.T

    hb = (jnp.dot(dist_ref[...], w1_ref[0:N_DE, :], preferred_element_type=_F32)
          + jnp.dot(pa_ref[...], w1_ref[N_DE:N_DE + N_AE, :],
                    preferred_element_type=_F32)
          + jnp.dot(pe, w1_ref[N_DE + N_AE:, :], preferred_element_type=_F32))

    scale = sl_ref[0:1, :]
    shift = sl_ref[1:2, :]
    out_ref[...] = jnp.maximum(hb * scale + shift + x_ref[...], 0.0)


@jax.jit
def kernel(x, dist_feat, atom_idx, ele_idx, W1, b1, gamma, beta):
    del b1
    aidx = atom_idx.astype(jnp.int32)
    eidx = ele_idx.astype(jnp.int32)
    lo = aidx[::B]
    hi = aidx[B - 1::B]
    aidx3 = aidx.reshape(NB, 1, B)
    eidx3 = eidx.reshape(NB, 1, B)
    aidx_p = jnp.pad(aidx, (0, NP - N_ROWS))

    grid_a = pltpu.PrefetchScalarGridSpec(
        num_scalar_prefetch=2,
        grid=(NB,),
        in_specs=[
            pl.BlockSpec((B, 64), lambda i, lo, hi: (i, 0)),
            pl.BlockSpec((B, N_DE), lambda i, lo, hi: (i, 0)),
            pl.BlockSpec((1, 1, B), lambda i, lo, hi: (i, 0, 0)),
            pl.BlockSpec((1, 1, B), lambda i, lo, hi: (i, 0, 0)),
        ],
        out_specs=[
            pl.BlockSpec((TR, FD), lambda i, lo, hi: (0, 0)),
            pl.BlockSpec((TE, FD), lambda i, lo, hi: (0, 0)),
            pl.BlockSpec((TR, TE), lambda i, lo, hi: (0, 0)),
            pl.BlockSpec((24, 16), lambda i, lo, hi: (0, 0)),
        ],
    )
    aacc, eacc, ctab, dd = pl.pallas_call(
        _k1a_body,
        grid_spec=grid_a,
        out_shape=[
            jax.ShapeDtypeStruct((TR, FD), _F32),
            jax.ShapeDtypeStruct((TE, FD), _F32),
            jax.ShapeDtypeStruct((TR, TE), _F32),
            jax.ShapeDtypeStruct((24, 16), _F32),
        ],
        compiler_params=pltpu.CompilerParams(
            dimension_semantics=("arbitrary",),
        ),
    )(lo, hi, x, dist_feat, aidx3, eidx3)

    sl, pa_tab = pl.pallas_call(
        _k1b_body,
        grid=(KC,),
        in_specs=[
            pl.BlockSpec((RC, FD), lambda i: (i, 0)),
            pl.BlockSpec((RC, TE), lambda i: (i, 0)),
            pl.BlockSpec((TE, FD), lambda i: (0, 0)),
            pl.BlockSpec((24, 16), lambda i: (0, 0)),
            pl.BlockSpec((80, 64), lambda i: (0, 0)),
            pl.BlockSpec((1, 64), lambda i: (0, 0)),
            pl.BlockSpec((1, 64), lambda i: (0, 0)),
        ],
        out_specs=[
            pl.BlockSpec((8, 64), lambda i: (0, 0)),
            pl.BlockSpec((RC, N_AE), lambda i: (i, 0)),
        ],
        out_shape=[
            jax.ShapeDtypeStruct((8, 64), _F32),
            jax.ShapeDtypeStruct((TR, N_AE), _F32),
        ],
        scratch_shapes=[pltpu.VMEM((88, 32), _F32)],
        compiler_params=pltpu.CompilerParams(
            dimension_semantics=("arbitrary",),
        ),
    )(aacc, ctab, eacc, dd, W1, gamma.reshape(1, 64), beta.reshape(1, 64))

    sc_gather = pl.kernel(
        _ksc_body,
        out_type=jax.ShapeDtypeStruct((NP, N_AE), _F32),
        mesh=plsc.VectorSubcoreMesh(core_axis_name="c", subcore_axis_name="s",
                                    num_cores=NC, num_subcores=NS),
        scratch_types=[
            pltpu.VMEM((CPW * GC,), jnp.int32),
            pltpu.VMEM((2, GC, N_AE), _F32),
            pltpu.SemaphoreType.DMA,
            pltpu.SemaphoreType.DMA,
        ],
        compiler_params=pltpu.CompilerParams(use_tc_tiling_on_sc=False),
    )
    parows = sc_gather(pa_tab, aidx_p)

    (out,) = pl.pallas_call(
        _k2_body,
        grid=(NB,),
        in_specs=[
            pl.BlockSpec((TE, FD), lambda i: (0, 0)),
            pl.BlockSpec((B, N_DE), lambda i: (i, 0)),
            pl.BlockSpec((B, N_AE), lambda i: (i, 0)),
            pl.BlockSpec((1, 1, B), lambda i: (i, 0, 0)),
            pl.BlockSpec((80, 64), lambda i: (0, 0)),
            pl.BlockSpec((B, 64), lambda i: (i, 0)),
            pl.BlockSpec((8, 64), lambda i: (0, 0)),
        ],
        out_specs=[
            pl.BlockSpec((B, 64), lambda i: (i, 0)),
        ],
        out_shape=[
            jax.ShapeDtypeStruct((N_ROWS, 64), jnp.float32),
        ],
        compiler_params=pltpu.CompilerParams(
            dimension_semantics=("arbitrary",),
        ),
    )(eacc, dist_feat, parows, eidx3, W1, x, sl)
    return out


# speedup vs baseline: 1.1009x; 1.0215x over previous
import jax
import jax.numpy as jnp
from jax import lax
from jax.experimental import pallas as pl
from jax.experimental.pallas import tpu as pltpu
from jax.experimental.pallas import tpu_sc as plsc

N_ROWS = 800000
N_AE = 32
N_DE = 16
N_SEG_ATOM = 50000

B = 1280
NB = N_ROWS // B
W = 128
TR = 50432
TE = 128
FS = 40
FD = 56
KC = 8
RC = TR // KC

NC = 2
NS = 16
NW = NC * NS
GC = 128
CPW = 196
NP = NW * CPW * GC

_C00 = (((0,), (0,)), ((), ()))
_C11 = (((1,), (1,)), ((), ()))
_F32 = jnp.float32


def _k1a_body(lo_ref, hi_ref, x_ref, dist_ref, aidx_ref, eidx_ref,
              aacc_ref, eacc_ref, c_ref, dd_ref):
    i = pl.program_id(0)

    @pl.when(i == 0)
    def _():
        aacc_ref[...] = jnp.zeros((TR, FD), _F32)
        eacc_ref[...] = jnp.zeros((TE, FD), _F32)
        c_ref[...] = jnp.zeros((TR, TE), _F32)
        dd_ref[...] = jnp.zeros((24, 16), _F32)

    aidx_row = aidx_ref[0]
    eidx_row = eidx_ref[0]

    dist = dist_ref[...]
    ones8 = jnp.ones((B, 8), _F32)
    x56a = jnp.concatenate([x_ref[:, 0:N_AE], ones8, dist], axis=1)
    x56e = jnp.concatenate([ones8, x_ref[:, N_AE:2 * N_AE], dist], axis=1)

    sub_e = lax.broadcasted_iota(jnp.int32, (TE, B), 0)
    ohe = (sub_e == eidx_row).astype(_F32)
    eacc_ref[...] += jnp.dot(ohe, x56e, preferred_element_type=_F32)

    dext = jnp.concatenate([dist, ones8], axis=1)
    dd_ref[...] += lax.dot_general(dext, dist, _C00,
                                   preferred_element_type=_F32)

    lo = lo_ref[i]
    hi = hi_ref[i]
    base = (lo // 8) * 8
    nwin = (hi - base) // W + 1
    sub_a = lax.broadcasted_iota(jnp.int32, (W, B), 0)
    rel0 = aidx_row - base

    def wloop(k, _):
        oh = (sub_a == (rel0 - k * W)).astype(_F32)
        aacc_ref[pl.ds(base + k * W, W), :] += jnp.dot(
            oh, x56a, preferred_element_type=_F32)
        c_ref[pl.ds(base + k * W, W), :] += lax.dot_general(
            oh, ohe, _C11, preferred_element_type=_F32)
        return 0

    lax.fori_loop(0, nwin, wloop, 0)


def _pooled(tab, cnt_col, val_cols):
    n = jnp.maximum(tab[:, cnt_col:cnt_col + 1], 1.0)
    p = jnp.maximum(tab[:, val_cols:val_cols + N_AE], 0.0) / n
    return n, p


def _k1b_body(aacc_ref, c_ref, eacc_ref, dd_ref, w1_ref, gamma_ref, beta_ref,
              sl_ref, pa_ref, acc_ref):
    i = pl.program_id(0)

    @pl.when(i == 0)
    def _():
        acc_ref[...] = jnp.zeros((88, 32), _F32)

    tab = aacc_ref[...]
    na, pa = _pooled(tab, N_AE, 0)
    pa_ref[...] = pa
    paw = pa * na
    onesc = jnp.ones((RC, 1), _F32)

    eacc = eacc_ref[...]
    ne, pe = _pooled(eacc, 0, 8)

    cpe = jnp.dot(c_ref[...], pe, preferred_element_type=_F32)

    acc_ref[0:32, :] += lax.dot_general(pa, paw, _C00,
                                        preferred_element_type=_F32)
    acc_ref[32:64, :] += lax.dot_general(pa, cpe, _C00,
                                         preferred_element_type=_F32)
    acc_ref[64:80, :] += lax.dot_general(tab[:, FS:FD], pa, _C00,
                                         preferred_element_type=_F32)
    acc_ref[80:81, :] += lax.dot_general(onesc, paw, _C00,
                                         preferred_element_type=_F32)

    @pl.when(i == KC - 1)
    def _():
        qa = acc_ref[0:32, :]
        gae = acc_ref[32:64, :]
        gda = acc_ref[64:80, :]
        sa = acc_ref[80:81, :]

        pew = pe * ne
        onese = jnp.ones((TE, 1), _F32)
        qe = lax.dot_general(pe, pew, _C00, preferred_element_type=_F32)
        gde = lax.dot_general(eacc[:, FS:FD], pe, _C00,
                              preferred_element_type=_F32)
        se = lax.dot_general(onese, pew, _C00, preferred_element_type=_F32)

        gdd = dd_ref[0:16, :]
        sd = dd_ref[16:17, :]

        w1d = w1_ref[0:N_DE, :]
        w1a = w1_ref[N_DE:N_DE + N_AE, :]
        w1e = w1_ref[N_DE + N_AE:, :]

        mu = (jnp.dot(sd, w1d, preferred_element_type=_F32)
              + jnp.dot(sa, w1a, preferred_element_type=_F32)
              + jnp.dot(se, w1e, preferred_element_type=_F32)) * (1.0 / N_ROWS)

        def dsum(a, m, b):
            return jnp.sum(a * jnp.dot(m, b, preferred_element_type=_F32),
                           axis=0, keepdims=True)

        hh = (dsum(w1d, gdd, w1d)
              + 2.0 * dsum(w1d, gda, w1a)
              + 2.0 * dsum(w1d, gde, w1e)
              + dsum(w1a, qa, w1a)
              + 2.0 * dsum(w1a, gae, w1e)
              + dsum(w1e, qe, w1e))

        var = hh * (1.0 / N_ROWS) - mu * mu
        scale = gamma_ref[...] * lax.rsqrt(var + 1e-5)
        shift = beta_ref[...] - mu * scale
        sl_ref[...] = jnp.concatenate(
            [scale, shift, jnp.zeros((6, 64), _F32)], axis=0)


NBUF = 4


def _ksc_body(pa_hbm, idx_hbm, out_hbm, idx_v, rows_v, *sems):
    wid = lax.axis_index("s") * NC + lax.axis_index("c")
    base0 = pl.multiple_of(wid * (CPW * GC), GC)
    pltpu.sync_copy(idx_hbm.at[pl.ds(base0, CPW * GC)], idx_v)

    def fire(c, b):
        pltpu.async_copy(
            pa_hbm.at[idx_v.at[pl.ds(pl.multiple_of(c * GC, GC), GC)]],
            rows_v.at[b], sems[b])

    for b in range(NBUF):
        fire(b, b)

    def step(g, carry):
        for b in range(NBUF):
            c = g * NBUF + b
            pltpu.make_async_copy(pa_hbm.at[idx_v.at[pl.ds(0, GC)]],
                                  rows_v.at[b], sems[b]).wait()
            pltpu.sync_copy(rows_v.at[b],
                            out_hbm.at[pl.ds(base0 + c * GC, GC)])

            @pl.when(c + NBUF < CPW)
            def _():
                fire(c + NBUF, b)
        return carry

    lax.fori_loop(0, CPW // NBUF, step, 0)


def _k2_body(eacc_ref, dist_ref, pa_ref, eidx_ref, w1_ref, x_ref, sl_ref,
             out_ref):
    eidx_row = eidx_ref[0]

    ecnt = jnp.maximum(eacc_ref[:, 0:1], 1.0)
    pe_tab = jnp.maximum(eacc_ref[:, 8:FS], 0.0) / ecnt
    sub_e = lax.broadcasted_iota(jnp.int32, (TE, B), 0)
    ohe = (sub_e == eidx_row).astype(_F32)
    pe_t = lax.dot_general(pe_tab, ohe, _C00, preferred_element_type=_F32)
    pe = pe_t.T

    hb = (jnp.dot(dist_ref[...], w1_ref[0:N_DE, :], preferred_element_type=_F32)
          + jnp.dot(pa_ref[...], w1_ref[N_DE:N_DE + N_AE, :],
                    preferred_element_type=_F32)
          + jnp.dot(pe, w1_ref[N_DE + N_AE:, :], preferred_element_type=_F32))

    scale = sl_ref[0:1, :]
    shift = sl_ref[1:2, :]
    out_ref[...] = jnp.maximum(hb * scale + shift + x_ref[...], 0.0)


@jax.jit
def kernel(x, dist_feat, atom_idx, ele_idx, W1, b1, gamma, beta):
    del b1
    aidx = atom_idx.astype(jnp.int32)
    eidx = ele_idx.astype(jnp.int32)
    lo = aidx[::B]
    hi = aidx[B - 1::B]
    aidx3 = aidx.reshape(NB, 1, B)
    eidx3 = eidx.reshape(NB, 1, B)
    aidx_p = jnp.pad(aidx, (0, NP - N_ROWS))

    grid_a = pltpu.PrefetchScalarGridSpec(
        num_scalar_prefetch=2,
        grid=(NB,),
        in_specs=[
            pl.BlockSpec((B, 64), lambda i, lo, hi: (i, 0)),
            pl.BlockSpec((B, N_DE), lambda i, lo, hi: (i, 0)),
            pl.BlockSpec((1, 1, B), lambda i, lo, hi: (i, 0, 0)),
            pl.BlockSpec((1, 1, B), lambda i, lo, hi: (i, 0, 0)),
        ],
        out_specs=[
            pl.BlockSpec((TR, FD), lambda i, lo, hi: (0, 0)),
            pl.BlockSpec((TE, FD), lambda i, lo, hi: (0, 0)),
            pl.BlockSpec((TR, TE), lambda i, lo, hi: (0, 0)),
            pl.BlockSpec((24, 16), lambda i, lo, hi: (0, 0)),
        ],
    )
    aacc, eacc, ctab, dd = pl.pallas_call(
        _k1a_body,
        grid_spec=grid_a,
        out_shape=[
            jax.ShapeDtypeStruct((TR, FD), _F32),
            jax.ShapeDtypeStruct((TE, FD), _F32),
            jax.ShapeDtypeStruct((TR, TE), _F32),
            jax.ShapeDtypeStruct((24, 16), _F32),
        ],
        compiler_params=pltpu.CompilerParams(
            dimension_semantics=("arbitrary",),
        ),
    )(lo, hi, x, dist_feat, aidx3, eidx3)

    sl, pa_tab = pl.pallas_call(
        _k1b_body,
        grid=(KC,),
        in_specs=[
            pl.BlockSpec((RC, FD), lambda i: (i, 0)),
            pl.BlockSpec((RC, TE), lambda i: (i, 0)),
            pl.BlockSpec((TE, FD), lambda i: (0, 0)),
            pl.BlockSpec((24, 16), lambda i: (0, 0)),
            pl.BlockSpec((80, 64), lambda i: (0, 0)),
            pl.BlockSpec((1, 64), lambda i: (0, 0)),
            pl.BlockSpec((1, 64), lambda i: (0, 0)),
        ],
        out_specs=[
            pl.BlockSpec((8, 64), lambda i: (0, 0)),
            pl.BlockSpec((RC, N_AE), lambda i: (i, 0)),
        ],
        out_shape=[
            jax.ShapeDtypeStruct((8, 64), _F32),
            jax.ShapeDtypeStruct((TR, N_AE), _F32),
        ],
        scratch_shapes=[pltpu.VMEM((88, 32), _F32)],
        compiler_params=pltpu.CompilerParams(
            dimension_semantics=("arbitrary",),
        ),
    )(aacc, ctab, eacc, dd, W1, gamma.reshape(1, 64), beta.reshape(1, 64))

    sc_gather = pl.kernel(
        _ksc_body,
        out_type=jax.ShapeDtypeStruct((NP, N_AE), _F32),
        mesh=plsc.VectorSubcoreMesh(core_axis_name="c", subcore_axis_name="s",
                                    num_cores=NC, num_subcores=NS),
        scratch_types=[
            pltpu.VMEM((CPW * GC,), jnp.int32),
            pltpu.VMEM((NBUF, GC, N_AE), _F32),
        ] + [pltpu.SemaphoreType.DMA] * NBUF,
        compiler_params=pltpu.CompilerParams(use_tc_tiling_on_sc=False),
    )
    parows = sc_gather(pa_tab, aidx_p)

    (out,) = pl.pallas_call(
        _k2_body,
        grid=(NB,),
        in_specs=[
            pl.BlockSpec((TE, FD), lambda i: (0, 0)),
            pl.BlockSpec((B, N_DE), lambda i: (i, 0)),
            pl.BlockSpec((B, N_AE), lambda i: (i, 0)),
            pl.BlockSpec((1, 1, B), lambda i: (i, 0, 0)),
            pl.BlockSpec((80, 64), lambda i: (0, 0)),
            pl.BlockSpec((B, 64), lambda i: (i, 0)),
            pl.BlockSpec((8, 64), lambda i: (0, 0)),
        ],
        out_specs=[
            pl.BlockSpec((B, 64), lambda i: (i, 0)),
        ],
        out_shape=[
            jax.ShapeDtypeStruct((N_ROWS, 64), jnp.float32),
        ],
        compiler_params=pltpu.CompilerParams(
            dimension_semantics=("arbitrary",),
        ),
    )(eacc, dist_feat, parows, eidx3, W1, x, sl)
    return out
